# baseline (speedup 1.0000x reference)
import functools

import jax
import jax.numpy as jnp
from jax import lax
from jax.experimental import pallas as pl
from jax.experimental.pallas import tpu as pltpu
from jax.experimental.pallas import tpu_sc as plsc

N = 10000
E = 320000
D = 128
H = 128

NC = 2
NS = 16
ETILE = E // NS

CH = 80
NCH = ETILE // CH
SLOTS = 5
ROUNDS = NCH // SLOTS

ROW_BLK = 80
NRB = N // ROW_BLK
RB_PER_SUB = (NRB + NS - 1) // NS

BLK = 2000
NBLK = N // BLK

ECHUNK = 2000
EPT = E // (NC * NS)
NECHUNK = EPT // ECHUNK

_MESH = plsc.VectorSubcoreMesh(core_axis_name="c", subcore_axis_name="s")
_SC_PARAMS = pltpu.CompilerParams(use_tc_tiling_on_sc=False)
_SC_GATHER_PARAMS = pltpu.CompilerParams(use_tc_tiling_on_sc=False,
                                         needs_layout_passes=False)


def _make_segsum(W):
    @functools.partial(
        pl.kernel,
        out_type=jax.ShapeDtypeStruct((NC * N, W), jnp.float32),
        mesh=_MESH,
        compiler_params=_SC_PARAMS,
        scratch_types=[
            pltpu.VMEM_SHARED((N, W), jnp.float32),
            pltpu.VMEM((2 * SLOTS, CH), jnp.int32),
            pltpu.VMEM((2 * SLOTS, CH), jnp.int32),
            pltpu.VMEM((2 * SLOTS, CH, W), jnp.float32),
            pltpu.VMEM((ROW_BLK, W), jnp.float32),
            pltpu.SemaphoreType.DMA((SLOTS,)),
            pltpu.SemaphoreType.DMA((SLOTS,)),
            pltpu.SemaphoreType.DMA((SLOTS,)),
        ],
    )
    def seg(tabl_hbm, tabr_hbm, src_hbm, dst_hbm, out_hbm,
            acc_sh, ibufs, dbufs, rbufs, zbuf, isem, gsem, ssem):
        c = lax.axis_index("c")
        s = lax.axis_index("s")
        base = s * ETILE

        def fire_idx(w, k):
            pltpu.async_copy(src_hbm.at[pl.ds(base + w * CH, CH)],
                             ibufs.at[k], isem.at[k % SLOTS])
            pltpu.async_copy(dst_hbm.at[pl.ds(base + w * CH, CH)],
                             dbufs.at[k], isem.at[k % SLOTS])

        def wait_idx(w, k):
            pltpu.make_async_copy(src_hbm.at[pl.ds(base + w * CH, CH)],
                                  ibufs.at[k], isem.at[k % SLOTS]).wait()
            pltpu.make_async_copy(dst_hbm.at[pl.ds(base + w * CH, CH)],
                                  dbufs.at[k], isem.at[k % SLOTS]).wait()

        def fire_gather(k):
            @pl.when(c == 0)
            def _():
                pltpu.async_copy(tabl_hbm.at[ibufs.at[k]], rbufs.at[k],
                                 gsem.at[k % SLOTS])

            @pl.when(c == 1)
            def _():
                pltpu.async_copy(tabr_hbm.at[ibufs.at[k]], rbufs.at[k],
                                 gsem.at[k % SLOTS])

        def wait_gather(k):
            pltpu.make_async_copy(tabl_hbm.at[ibufs.at[k]], rbufs.at[k],
                                  gsem.at[k % SLOTS]).wait()

        def fire_scatter(k):
            pltpu.async_copy(rbufs.at[k], acc_sh.at[dbufs.at[k]],
                             ssem.at[k % SLOTS], add=True)

        def wait_scatter(k):
            pltpu.make_async_copy(rbufs.at[k], acc_sh.at[dbufs.at[k]],
                                  ssem.at[k % SLOTS]).wait()

        for r in range(SLOTS):
            fire_idx(r, 2 * r)

        @pl.loop(0, ROW_BLK)
        def _zr(r):
            @pl.loop(0, W, step=16)
            def _zc(j):
                zbuf[r, pl.ds(j, 16)] = jnp.zeros((16,), jnp.float32)

        @pl.loop(0, RB_PER_SUB)
        def _zb(k):
            b = s * RB_PER_SUB + k

            @pl.when(b < NRB)
            def _():
                pltpu.sync_copy(zbuf, acc_sh.at[pl.ds(b * ROW_BLK, ROW_BLK)])

        plsc.subcore_barrier()

        def round_body(i, p, wait_prev, fire_next):
            for r in range(SLOTS):
                w = i * SLOTS + r
                wait_idx(w, 2 * r + p)
                fire_gather(2 * r + p)
            for r in range(SLOTS):
                w = i * SLOTS + r
                if wait_prev:
                    wait_scatter(2 * r + (1 - p))
                if fire_next:
                    fire_idx(w + SLOTS, 2 * r + (1 - p))
            for r in range(SLOTS):
                wait_gather(2 * r + p)
                fire_scatter(2 * r + p)

        round_body(0, 0, wait_prev=False, fire_next=True)
        round_body(1, 1, wait_prev=True, fire_next=True)

        @pl.loop(0, (ROUNDS - 4) // 2)
        def _main(j):
            round_body(2 + 2 * j, 0, wait_prev=True, fire_next=True)
            round_body(3 + 2 * j, 1, wait_prev=True, fire_next=True)

        round_body(ROUNDS - 2, 0, wait_prev=True, fire_next=True)
        round_body(ROUNDS - 1, 1, wait_prev=True, fire_next=False)
        for r in range(SLOTS):
            wait_scatter(2 * r + 1)

        plsc.subcore_barrier()

        @pl.loop(0, RB_PER_SUB)
        def _wb(k):
            b = s * RB_PER_SUB + k

            @pl.when(b < NRB)
            def _():
                pltpu.sync_copy(acc_sh.at[pl.ds(b * ROW_BLK, ROW_BLK)],
                                out_hbm.at[pl.ds(c * N + b * ROW_BLK, ROW_BLK)])

    return seg


_sc_segsum80 = _make_segsum(80)
_sc_segsum64 = _make_segsum(64)


@functools.partial(
    pl.kernel,
    out_type=jax.ShapeDtypeStruct((2 * E,), jnp.float32),
    mesh=_MESH,
    compiler_params=_SC_GATHER_PARAMS,
    scratch_types=[
        pltpu.VMEM((N, 4), jnp.float32),
        pltpu.VMEM((ECHUNK,), jnp.int32),
        pltpu.VMEM((ECHUNK,), jnp.int32),
        pltpu.VMEM((2 * ECHUNK,), jnp.float32),
    ],
)
def _sc_edge_logits(pq_hbm, src_hbm, dst_hbm, out_hbm,
                    pq_v, sbuf, dbuf, obuf):
    c = lax.axis_index("c")
    s = lax.axis_index("s")
    pltpu.sync_copy(pq_hbm, pq_v)
    base = (c * NS + s) * EPT
    col0 = jnp.full((16,), 0, jnp.int32)
    col1 = jnp.full((16,), 1, jnp.int32)
    col2 = jnp.full((16,), 2, jnp.int32)
    col3 = jnp.full((16,), 3, jnp.int32)
    lane2 = lax.iota(jnp.int32, 16) * 2

    @pl.loop(0, NECHUNK)
    def _win(i):
        off = base + i * ECHUNK
        pltpu.sync_copy(src_hbm.at[pl.ds(off, ECHUNK)], sbuf)
        pltpu.sync_copy(dst_hbm.at[pl.ds(off, ECHUNK)], dbuf)

        @pl.loop(0, ECHUNK // 16)
        def _vec(j):
            sv = sbuf[pl.ds(j * 16, 16)]
            dv = dbuf[pl.ds(j * 16, 16)]
            l0 = (plsc.load_gather(pq_v, [sv, col0])
                  + plsc.load_gather(pq_v, [dv, col2]))
            l1 = (plsc.load_gather(pq_v, [sv, col1])
                  + plsc.load_gather(pq_v, [dv, col3]))
            pos = lane2 + j * 32
            plsc.store_scatter(obuf, [pos], l0)
            plsc.store_scatter(obuf, [pos + 1], l1)

        pltpu.sync_copy(obuf, out_hbm.at[pl.ds(2 * off, 2 * ECHUNK)])


def _layer_body(pl_ref, pr_ref, x_ref, wl_ref, b_ref, wr_ref,
                out_ref, tl_ref, tr_ref):
    summed = jnp.concatenate([pl_ref[...], pr_ref[:, :48]], axis=1)
    deg = jnp.maximum(pr_ref[:, 48:49], 1.0)
    mean = summed / deg
    h = (lax.dot_general(mean, wl_ref[...], (((1,), (1,)), ((), ())),
                         preferred_element_type=jnp.float32,
                         precision=lax.Precision.HIGHEST)
         + b_ref[...]
         + lax.dot_general(x_ref[...], wr_ref[...], (((1,), (1,)), ((), ())),
                           preferred_element_type=jnp.float32,
                           precision=lax.Precision.HIGHEST))
    h = jnp.maximum(h, 0.0)
    out_ref[...] = h
    tl_ref[...] = h[:, :64]
    tr_ref[...] = h[:, 64:]


def _tc_layer(acc, x, Wl, b, Wr):
    return pl.pallas_call(
        _layer_body,
        grid=(NBLK,),
        in_specs=[
            pl.BlockSpec((BLK, 80), lambda i: (i, 0)),
            pl.BlockSpec((BLK, 80), lambda i: (i + NBLK, 0)),
            pl.BlockSpec((BLK, D), lambda i: (i, 0)),
            pl.BlockSpec((H, D), lambda i: (0, 0)),
            pl.BlockSpec((1, H), lambda i: (0, 0)),
            pl.BlockSpec((H, D), lambda i: (0, 0)),
        ],
        out_specs=[
            pl.BlockSpec((BLK, H), lambda i: (i, 0)),
            pl.BlockSpec((BLK, 64), lambda i: (i, 0)),
            pl.BlockSpec((BLK, 64), lambda i: (i, 0)),
        ],
        out_shape=[
            jax.ShapeDtypeStruct((N, H), jnp.float32),
            jax.ShapeDtypeStruct((N, 64), jnp.float32),
            jax.ShapeDtypeStruct((N, 64), jnp.float32),
        ],
    )(acc, acc, x, Wl, b.reshape(1, H), Wr)


def _pq_body(pl_ref, pr_ref, deg_ref, h_ref, wl_ref, b_ref, wr_ref,
             w4_ref, bc4_ref, out_ref):
    summed = jnp.concatenate([pl_ref[...], pr_ref[...]], axis=1)
    deg = jnp.maximum(deg_ref[...], 1.0)
    mean = summed / deg
    h = (lax.dot_general(mean, wl_ref[...], (((1,), (1,)), ((), ())),
                         preferred_element_type=jnp.float32,
                         precision=lax.Precision.HIGHEST)
         + b_ref[...]
         + lax.dot_general(h_ref[...], wr_ref[...], (((1,), (1,)), ((), ())),
                           preferred_element_type=jnp.float32,
                           precision=lax.Precision.HIGHEST))
    h = jnp.maximum(h, 0.0)
    out_ref[...] = lax.dot_general(h, w4_ref[...], (((1,), (1,)), ((), ())),
                                   preferred_element_type=jnp.float32,
                                   precision=lax.Precision.HIGHEST) + bc4_ref[...]


def _tc_pq(acc2, deg, h1, Wl, b, Wr, W4, bc4):
    return pl.pallas_call(
        _pq_body,
        grid=(NBLK,),
        in_specs=[
            pl.BlockSpec((BLK, 64), lambda i: (i, 0)),
            pl.BlockSpec((BLK, 64), lambda i: (i + NBLK, 0)),
            pl.BlockSpec((BLK, 1), lambda i: (i, 0)),
            pl.BlockSpec((BLK, H), lambda i: (i, 0)),
            pl.BlockSpec((H, H), lambda i: (0, 0)),
            pl.BlockSpec((1, H), lambda i: (0, 0)),
            pl.BlockSpec((H, H), lambda i: (0, 0)),
            pl.BlockSpec((4, H), lambda i: (0, 0)),
            pl.BlockSpec((1, 4), lambda i: (0, 0)),
        ],
        out_specs=pl.BlockSpec((BLK, 4), lambda i: (i, 0)),
        out_shape=jax.ShapeDtypeStruct((N, 4), jnp.float32),
    )(acc2, acc2, deg, h1, Wl, b.reshape(1, H), Wr, W4, bc4.reshape(1, 4))


def kernel(x, edge_index, W1l, b1, W1r, W2l, b2, W2r, Wc, bc):
    src = edge_index[0]
    dst = edge_index[1]
    tab1l = x[:, :80]
    tab1r = jnp.concatenate([x[:, 80:], jnp.ones((N, 1), jnp.float32),
                             jnp.zeros((N, 31), jnp.float32)], axis=1)

    acc1 = _sc_segsum80(tab1l, tab1r, src, dst)
    h1, tab2l, tab2r = _tc_layer(acc1, x, W1l, b1, W1r)

    acc2 = _sc_segsum64(tab2l, tab2r, src, dst)

    deg = acc1[N:, 48:49]
    W4 = jnp.concatenate([Wc[:, :H], Wc[:, H:]], axis=0)
    bc4 = jnp.concatenate([bc, jnp.zeros((2,), jnp.float32)])
    pq = _tc_pq(acc2, deg, h1, W2l, b2, W2r, W4, bc4)

    flat = _sc_edge_logits(pq, src, dst)
    return flat.reshape(E, 2)


# speedup vs baseline: 1.5254x; 1.5254x over previous
import functools

import jax
import jax.numpy as jnp
from jax import lax
from jax.experimental import pallas as pl
from jax.experimental.pallas import tpu as pltpu
from jax.experimental.pallas import tpu_sc as plsc

N = 10000
E = 320000
D = 128
H = 128

NC = 2
NS = 16
ETILE = E // NS

CH = 80
NCH = ETILE // CH
SLOTS = 5
ROUNDS = NCH // SLOTS

ROW_BLK = 80
NRB = N // ROW_BLK
RB_PER_SUB = (NRB + NS - 1) // NS

BLK = 2000
NBLK = N // BLK

ECHUNK = 2000
EPT = E // (NC * NS)
NECHUNK = EPT // ECHUNK

_MESH = plsc.VectorSubcoreMesh(core_axis_name="c", subcore_axis_name="s")
_SC_PARAMS = pltpu.CompilerParams(use_tc_tiling_on_sc=False,
                                  skip_device_barrier=True)
_SC_GATHER_PARAMS = pltpu.CompilerParams(use_tc_tiling_on_sc=False,
                                         needs_layout_passes=False,
                                         skip_device_barrier=True)


def _make_segsum(W):
    @functools.partial(
        pl.kernel,
        out_type=jax.ShapeDtypeStruct((NC * N, W), jnp.float32),
        mesh=_MESH,
        compiler_params=_SC_PARAMS,
        scratch_types=[
            pltpu.VMEM_SHARED((N, W), jnp.float32),
            pltpu.VMEM((2 * SLOTS, CH), jnp.int32),
            pltpu.VMEM((2 * SLOTS, CH), jnp.int32),
            pltpu.VMEM((2 * SLOTS, CH, W), jnp.float32),
            pltpu.VMEM((ROW_BLK, W), jnp.float32),
            pltpu.SemaphoreType.DMA((SLOTS,)),
            pltpu.SemaphoreType.DMA((SLOTS,)),
            pltpu.SemaphoreType.DMA((SLOTS,)),
        ],
    )
    def seg(table_hbm, srcs_hbm, dst_hbm, out_hbm,
            acc_sh, ibufs, dbufs, rbufs, zbuf, isem, gsem, ssem):
        c = lax.axis_index("c")
        s = lax.axis_index("s")
        cbase = c * E + s * ETILE
        sbase = s * ETILE

        def fire_idx(w, k):
            pltpu.async_copy(srcs_hbm.at[pl.ds(cbase + w * CH, CH)],
                             ibufs.at[k], isem.at[k % SLOTS])
            pltpu.async_copy(dst_hbm.at[pl.ds(sbase + w * CH, CH)],
                             dbufs.at[k], isem.at[k % SLOTS])

        def wait_idx(w, k):
            pltpu.make_async_copy(srcs_hbm.at[pl.ds(cbase + w * CH, CH)],
                                  ibufs.at[k], isem.at[k % SLOTS]).wait()
            pltpu.make_async_copy(dst_hbm.at[pl.ds(sbase + w * CH, CH)],
                                  dbufs.at[k], isem.at[k % SLOTS]).wait()

        def fire_gather(k):
            pltpu.async_copy(table_hbm.at[ibufs.at[k]], rbufs.at[k],
                             gsem.at[k % SLOTS])

        def wait_gather(k):
            pltpu.make_async_copy(table_hbm.at[ibufs.at[k]], rbufs.at[k],
                                  gsem.at[k % SLOTS]).wait()

        def fire_scatter(k):
            pltpu.async_copy(rbufs.at[k], acc_sh.at[dbufs.at[k]],
                             ssem.at[k % SLOTS], add=True)

        def wait_scatter(k):
            pltpu.make_async_copy(rbufs.at[k], acc_sh.at[dbufs.at[k]],
                                  ssem.at[k % SLOTS]).wait()

        for r in range(SLOTS):
            fire_idx(r, 2 * r)

        @pl.loop(0, ROW_BLK)
        def _zr(r):
            @pl.loop(0, W, step=16)
            def _zc(j):
                zbuf[r, pl.ds(j, 16)] = jnp.zeros((16,), jnp.float32)

        @pl.loop(0, RB_PER_SUB)
        def _zb(k):
            b = s * RB_PER_SUB + k

            @pl.when(b < NRB)
            def _():
                pltpu.sync_copy(zbuf, acc_sh.at[pl.ds(b * ROW_BLK, ROW_BLK)])

        plsc.subcore_barrier()

        def round_body(i, p, wait_prev, fire_next):
            for r in range(SLOTS):
                w = i * SLOTS + r
                wait_idx(w, 2 * r + p)
                fire_gather(2 * r + p)
            for r in range(SLOTS):
                w = i * SLOTS + r
                if wait_prev:
                    wait_scatter(2 * r + (1 - p))
                if fire_next:
                    fire_idx(w + SLOTS, 2 * r + (1 - p))
            for r in range(SLOTS):
                wait_gather(2 * r + p)
                fire_scatter(2 * r + p)

        round_body(0, 0, wait_prev=False, fire_next=True)
        round_body(1, 1, wait_prev=True, fire_next=True)

        @pl.loop(0, (ROUNDS - 4) // 2)
        def _main(j):
            round_body(2 + 2 * j, 0, wait_prev=True, fire_next=True)
            round_body(3 + 2 * j, 1, wait_prev=True, fire_next=True)

        round_body(ROUNDS - 2, 0, wait_prev=True, fire_next=True)
        round_body(ROUNDS - 1, 1, wait_prev=True, fire_next=False)
        for r in range(SLOTS):
            wait_scatter(2 * r + 1)

        plsc.subcore_barrier()

        @pl.loop(0, RB_PER_SUB)
        def _wb(k):
            b = s * RB_PER_SUB + k

            @pl.when(b < NRB)
            def _():
                pltpu.sync_copy(acc_sh.at[pl.ds(b * ROW_BLK, ROW_BLK)],
                                out_hbm.at[pl.ds(c * N + b * ROW_BLK, ROW_BLK)])

    return seg


_sc_segsum80 = _make_segsum(80)
_sc_segsum64 = _make_segsum(64)


@functools.partial(
    pl.kernel,
    out_type=jax.ShapeDtypeStruct((2 * E,), jnp.float32),
    mesh=_MESH,
    compiler_params=_SC_GATHER_PARAMS,
    scratch_types=[
        pltpu.VMEM((N, 4), jnp.float32),
        pltpu.VMEM((ECHUNK,), jnp.int32),
        pltpu.VMEM((ECHUNK,), jnp.int32),
        pltpu.VMEM((ECHUNK,), jnp.float32),
        pltpu.VMEM((ECHUNK,), jnp.float32),
    ],
)
def _sc_edge_logits(pq_hbm, src_hbm, dst_hbm, out_hbm,
                    pq_v, sbuf, dbuf, o0, o1):
    c = lax.axis_index("c")
    s = lax.axis_index("s")
    pltpu.sync_copy(pq_hbm, pq_v)
    base = (c * NS + s) * EPT
    col0 = jnp.full((16,), 0, jnp.int32)
    col1 = jnp.full((16,), 1, jnp.int32)
    col2 = jnp.full((16,), 2, jnp.int32)
    col3 = jnp.full((16,), 3, jnp.int32)

    @pl.loop(0, NECHUNK)
    def _win(i):
        off = base + i * ECHUNK
        pltpu.sync_copy(src_hbm.at[pl.ds(off, ECHUNK)], sbuf)
        pltpu.sync_copy(dst_hbm.at[pl.ds(off, ECHUNK)], dbuf)

        @pl.loop(0, ECHUNK // 16)
        def _vec(j):
            sv = sbuf[pl.ds(j * 16, 16)]
            dv = dbuf[pl.ds(j * 16, 16)]
            l0 = (plsc.load_gather(pq_v, [sv, col0])
                  + plsc.load_gather(pq_v, [dv, col2]))
            l1 = (plsc.load_gather(pq_v, [sv, col1])
                  + plsc.load_gather(pq_v, [dv, col3]))
            o0[pl.ds(j * 16, 16)] = l0
            o1[pl.ds(j * 16, 16)] = l1

        pltpu.sync_copy(o0, out_hbm.at[pl.ds(off, ECHUNK)])
        pltpu.sync_copy(o1, out_hbm.at[pl.ds(E + off, ECHUNK)])


def _layer_body(pl_ref, pr_ref, x_ref, wl_ref, b_ref, wr_ref, out_ref):
    summed = jnp.concatenate([pl_ref[...], pr_ref[:, :48]], axis=1)
    deg = jnp.maximum(pr_ref[:, 48:49], 1.0)
    mean = summed / deg
    h = (lax.dot_general(mean, wl_ref[...], (((1,), (1,)), ((), ())),
                         preferred_element_type=jnp.float32,
                         precision=lax.Precision.HIGHEST)
         + b_ref[...]
         + lax.dot_general(x_ref[...], wr_ref[...], (((1,), (1,)), ((), ())),
                           preferred_element_type=jnp.float32,
                           precision=lax.Precision.HIGHEST))
    out_ref[...] = jnp.maximum(h, 0.0)


def _tc_layer(acc, x, Wl, b, Wr):
    return pl.pallas_call(
        _layer_body,
        grid=(NBLK,),
        in_specs=[
            pl.BlockSpec((BLK, 80), lambda i: (i, 0)),
            pl.BlockSpec((BLK, 80), lambda i: (i + NBLK, 0)),
            pl.BlockSpec((BLK, D), lambda i: (i, 0)),
            pl.BlockSpec((H, D), lambda i: (0, 0)),
            pl.BlockSpec((1, H), lambda i: (0, 0)),
            pl.BlockSpec((H, D), lambda i: (0, 0)),
        ],
        out_specs=pl.BlockSpec((BLK, H), lambda i: (i, 0)),
        out_shape=jax.ShapeDtypeStruct((N, H), jnp.float32),
    )(acc, acc, x, Wl, b.reshape(1, H), Wr)


def _pq_body(pl_ref, pr_ref, deg_ref, h_ref, wl_ref, b_ref, wr_ref,
             w4_ref, bc4_ref, out_ref):
    summed = jnp.concatenate([pl_ref[...], pr_ref[...]], axis=1)
    deg = jnp.maximum(deg_ref[...], 1.0)
    mean = summed / deg
    h = (lax.dot_general(mean, wl_ref[...], (((1,), (1,)), ((), ())),
                         preferred_element_type=jnp.float32,
                         precision=lax.Precision.HIGHEST)
         + b_ref[...]
         + lax.dot_general(h_ref[...], wr_ref[...], (((1,), (1,)), ((), ())),
                           preferred_element_type=jnp.float32,
                           precision=lax.Precision.HIGHEST))
    h = jnp.maximum(h, 0.0)
    out_ref[...] = lax.dot_general(h, w4_ref[...], (((1,), (1,)), ((), ())),
                                   preferred_element_type=jnp.float32,
                                   precision=lax.Precision.HIGHEST) + bc4_ref[...]


def _tc_pq(acc2, deg, h1, Wl, b, Wr, W4, bc4):
    return pl.pallas_call(
        _pq_body,
        grid=(NBLK,),
        in_specs=[
            pl.BlockSpec((BLK, 64), lambda i: (i, 0)),
            pl.BlockSpec((BLK, 64), lambda i: (i + NBLK, 0)),
            pl.BlockSpec((BLK, 1), lambda i: (i, 0)),
            pl.BlockSpec((BLK, H), lambda i: (i, 0)),
            pl.BlockSpec((H, H), lambda i: (0, 0)),
            pl.BlockSpec((1, H), lambda i: (0, 0)),
            pl.BlockSpec((H, H), lambda i: (0, 0)),
            pl.BlockSpec((4, H), lambda i: (0, 0)),
            pl.BlockSpec((1, 4), lambda i: (0, 0)),
        ],
        out_specs=pl.BlockSpec((BLK, 4), lambda i: (i, 0)),
        out_shape=jax.ShapeDtypeStruct((N, 4), jnp.float32),
    )(acc2, acc2, deg, h1, Wl, b.reshape(1, H), Wr, W4, bc4.reshape(1, 4))


def kernel(x, edge_index, W1l, b1, W1r, W2l, b2, W2r, Wc, bc):
    src = edge_index[0]
    dst = edge_index[1]
    srcs = jnp.concatenate([src, src + N])
    tab1 = jnp.concatenate(
        [x[:, :80],
         jnp.concatenate([x[:, 80:], jnp.ones((N, 1), jnp.float32),
                          jnp.zeros((N, 31), jnp.float32)], axis=1)],
        axis=0)

    acc1 = _sc_segsum80(tab1, srcs, dst)
    h1 = _tc_layer(acc1, x, W1l, b1, W1r)

    tab2 = jnp.concatenate([h1[:, :64], h1[:, 64:]], axis=0)
    acc2 = _sc_segsum64(tab2, srcs, dst)

    deg = acc1[N:, 48:49]
    W4 = jnp.concatenate([Wc[:, :H], Wc[:, H:]], axis=0)
    bc4 = jnp.concatenate([bc, jnp.zeros((2,), jnp.float32)])
    pq = _tc_pq(acc2, deg, h1, W2l, b2, W2r, W4, bc4)

    flat = _sc_edge_logits(pq, src, dst)
    return flat.reshape(2, E).T
